# baseline (speedup 1.0000x reference)
import jax
import jax.numpy as jnp
from jax.experimental import pallas as pl
from jax.experimental.pallas import tpu as pltpu

_TN = 1024
_NBUF = 6


def _attn_pipeline(q_hbm, k_ref, v_ref, o_hbm, qbuf, obuf, insem, outsem):
    total = q_hbm.shape[0]
    ntiles = total // _TN
    rows_per_batch = 4096

    def start_in(t):
        slot = t % _NBUF
        pltpu.make_async_copy(
            q_hbm.at[pl.ds(t * _TN, _TN), :], qbuf.at[slot], insem.at[slot]
        ).start()

    for t in range(min(_NBUF, ntiles)):
        start_in(t)

    for t in range(ntiles):
        slot = t % _NBUF
        pltpu.make_async_copy(
            q_hbm.at[pl.ds(t * _TN, _TN), :], qbuf.at[slot], insem.at[slot]
        ).wait()
        if t >= _NBUF:
            pltpu.make_async_copy(
                obuf.at[slot], o_hbm.at[pl.ds((t - _NBUF) * _TN, _TN), :],
                outsem.at[slot],
            ).wait()
        b = (t * _TN) // rows_per_batch
        k64 = k_ref[b]
        v64 = v_ref[b]
        s = jax.lax.dot_general(
            qbuf[slot], k64, (((1,), (1,)), ((), ())),
            preferred_element_type=jnp.float32,
            precision=jax.lax.Precision.DEFAULT,
        )
        m = jnp.max(s, axis=-1, keepdims=True)
        e = jnp.exp(s - m)
        p = e / jnp.sum(e, axis=-1, keepdims=True)
        obuf[slot] = jax.lax.dot_general(
            p, v64, (((1,), (0,)), ((), ())),
            preferred_element_type=jnp.float32,
            precision=jax.lax.Precision.DEFAULT,
        )
        pltpu.make_async_copy(
            obuf.at[slot], o_hbm.at[pl.ds(t * _TN, _TN), :], outsem.at[slot]
        ).start()
        if t + _NBUF < ntiles:
            start_in(t + _NBUF)

    for t in range(max(0, ntiles - _NBUF), ntiles):
        slot = t % _NBUF
        pltpu.make_async_copy(
            obuf.at[slot], o_hbm.at[pl.ds(t * _TN, _TN), :], outsem.at[slot]
        ).wait()


def kernel(q, k, v, lsh_projection):
    B, N, D = q.shape
    H = lsh_projection.shape[0]
    k64 = k[:, :H, :]
    v64 = v[:, :H, :]
    qf = q.reshape(B * N, D)
    out = pl.pallas_call(
        _attn_pipeline,
        grid=(),
        in_specs=[
            pl.BlockSpec(memory_space=pltpu.MemorySpace.HBM),
            pl.BlockSpec(memory_space=pltpu.MemorySpace.VMEM),
            pl.BlockSpec(memory_space=pltpu.MemorySpace.VMEM),
        ],
        out_specs=pl.BlockSpec(memory_space=pltpu.MemorySpace.HBM),

        out_shape=jax.ShapeDtypeStruct((B * N, D), jnp.float32),
        scratch_shapes=[
            pltpu.VMEM((_NBUF, _TN, D), jnp.float32),
            pltpu.VMEM((_NBUF, _TN, D), jnp.float32),
            pltpu.SemaphoreType.DMA((_NBUF,)),
            pltpu.SemaphoreType.DMA((_NBUF,)),
        ],
    )(qf, k64, v64)
    return out.reshape(B, N, D)


# speedup vs baseline: 1.0165x; 1.0165x over previous
import jax
import jax.numpy as jnp
from jax.experimental import pallas as pl
from jax.experimental.pallas import tpu as pltpu

_TN = 2048
_NBUF = 3
_NCHUNK = 2
_TC = _TN // _NCHUNK


def _attn_pipeline(q_hbm, k_ref, v_ref, o_hbm, qbuf, obuf, insem, outsem):
    total = q_hbm.shape[0]
    ntiles = total // _TN
    rows_per_batch = 4096

    def start_in(t):
        slot = t % _NBUF
        pltpu.make_async_copy(
            q_hbm.at[pl.ds(t * _TN, _TN), :], qbuf.at[slot], insem.at[slot]
        ).start()

    def out_copy(t, c):
        slot = t % _NBUF
        return pltpu.make_async_copy(
            obuf.at[slot, c],
            o_hbm.at[pl.ds(t * _TN + c * _TC, _TC), :],
            outsem.at[slot, c],
        )

    for t in range(min(_NBUF, ntiles)):
        start_in(t)

    for t in range(ntiles):
        slot = t % _NBUF
        pltpu.make_async_copy(
            q_hbm.at[pl.ds(t * _TN, _TN), :], qbuf.at[slot], insem.at[slot]
        ).wait()
        b = (t * _TN) // rows_per_batch
        k64 = k_ref[b]
        v64 = v_ref[b]
        for c in range(_NCHUNK):
            if t >= _NBUF:
                out_copy(t - _NBUF, c).wait()
            s = jax.lax.dot_general(
                qbuf[slot, pl.ds(c * _TC, _TC), :], k64,
                (((1,), (1,)), ((), ())),
                preferred_element_type=jnp.float32,
                precision=jax.lax.Precision.DEFAULT,
            )
            m = jnp.max(s, axis=-1, keepdims=True)
            e = jnp.exp(s - m)
            p = e / jnp.sum(e, axis=-1, keepdims=True)
            obuf[slot, c] = jax.lax.dot_general(
                p, v64, (((1,), (0,)), ((), ())),
                preferred_element_type=jnp.float32,
                precision=jax.lax.Precision.DEFAULT,
            )
            out_copy(t, c).start()
        if t + _NBUF < ntiles:
            start_in(t + _NBUF)

    for t in range(max(0, ntiles - _NBUF), ntiles):
        for c in range(_NCHUNK):
            out_copy(t, c).wait()


def kernel(q, k, v, lsh_projection):
    B, N, D = q.shape
    H = lsh_projection.shape[0]
    k64 = k[:, :H, :]
    v64 = v[:, :H, :]
    qf = q.reshape(B * N, D)
    out = pl.pallas_call(
        _attn_pipeline,
        grid=(),
        in_specs=[
            pl.BlockSpec(memory_space=pltpu.MemorySpace.HBM),
            pl.BlockSpec(memory_space=pltpu.MemorySpace.VMEM),
            pl.BlockSpec(memory_space=pltpu.MemorySpace.VMEM),
        ],
        out_specs=pl.BlockSpec(memory_space=pltpu.MemorySpace.HBM),
        out_shape=jax.ShapeDtypeStruct((B * N, D), jnp.float32),
        scratch_shapes=[
            pltpu.VMEM((_NBUF, _TN, D), jnp.float32),
            pltpu.VMEM((_NBUF, _NCHUNK, _TC, D), jnp.float32),
            pltpu.SemaphoreType.DMA((_NBUF,)),
            pltpu.SemaphoreType.DMA((_NBUF, _NCHUNK)),
        ],
    )(qf, k64, v64)
    return out.reshape(B, N, D)


# speedup vs baseline: 1.0994x; 1.0816x over previous
import jax
import jax.numpy as jnp
from jax.experimental import pallas as pl
from jax.experimental.pallas import tpu as pltpu

_TN = 2048
_NBUF = 3


def _attn_pipeline(q_hbm, k_hbm, v_hbm, o_hbm, qbuf, obuf, kbuf, vbuf,
                   insem, outsem, kvsem):
    total = q_hbm.shape[0]
    ntiles = total // _TN
    nbatch = k_hbm.shape[0]
    rows_per_batch = total // nbatch
    H = kbuf.shape[1]

    def start_in(t):
        slot = t % _NBUF
        pltpu.make_async_copy(
            q_hbm.at[pl.ds(t * _TN, _TN), :], qbuf.at[slot], insem.at[slot]
        ).start()

    kv_copies = []
    for b in range(nbatch):
        kv_copies.append(pltpu.make_async_copy(
            k_hbm.at[b, pl.ds(0, H), :], kbuf.at[b], kvsem.at[0, b]))
        kv_copies.append(pltpu.make_async_copy(
            v_hbm.at[b, pl.ds(0, H), :], vbuf.at[b], kvsem.at[1, b]))
    for c in kv_copies:
        c.start()

    for t in range(min(_NBUF, ntiles)):
        start_in(t)

    for c in kv_copies:
        c.wait()

    for t in range(ntiles):
        slot = t % _NBUF
        pltpu.make_async_copy(
            q_hbm.at[pl.ds(t * _TN, _TN), :], qbuf.at[slot], insem.at[slot]
        ).wait()
        if t >= _NBUF:
            pltpu.make_async_copy(
                obuf.at[slot], o_hbm.at[pl.ds((t - _NBUF) * _TN, _TN), :],
                outsem.at[slot],
            ).wait()
        b = (t * _TN) // rows_per_batch
        k64 = kbuf[b]
        v64 = vbuf[b]
        s = jax.lax.dot_general(
            qbuf[slot], k64, (((1,), (1,)), ((), ())),
            preferred_element_type=jnp.float32,
            precision=jax.lax.Precision.DEFAULT,
        )
        m = jnp.max(s, axis=-1, keepdims=True)
        e = jnp.exp(s - m)
        p = e / jnp.sum(e, axis=-1, keepdims=True)
        obuf[slot] = jax.lax.dot_general(
            p, v64, (((1,), (0,)), ((), ())),
            preferred_element_type=jnp.float32,
            precision=jax.lax.Precision.DEFAULT,
        )
        pltpu.make_async_copy(
            obuf.at[slot], o_hbm.at[pl.ds(t * _TN, _TN), :], outsem.at[slot]
        ).start()
        if t + _NBUF < ntiles:
            start_in(t + _NBUF)

    for t in range(max(0, ntiles - _NBUF), ntiles):
        slot = t % _NBUF
        pltpu.make_async_copy(
            obuf.at[slot], o_hbm.at[pl.ds(t * _TN, _TN), :], outsem.at[slot]
        ).wait()


def kernel(q, k, v, lsh_projection):
    B, N, D = q.shape
    H = lsh_projection.shape[0]
    qf = q.reshape(B * N, D)
    out = pl.pallas_call(
        _attn_pipeline,
        grid=(),
        in_specs=[
            pl.BlockSpec(memory_space=pltpu.MemorySpace.HBM),
            pl.BlockSpec(memory_space=pltpu.MemorySpace.HBM),
            pl.BlockSpec(memory_space=pltpu.MemorySpace.HBM),
        ],
        out_specs=pl.BlockSpec(memory_space=pltpu.MemorySpace.HBM),
        out_shape=jax.ShapeDtypeStruct((B * N, D), jnp.float32),
        scratch_shapes=[
            pltpu.VMEM((_NBUF, _TN, D), jnp.float32),
            pltpu.VMEM((_NBUF, _TN, D), jnp.float32),
            pltpu.VMEM((B, H, D), jnp.float32),
            pltpu.VMEM((B, H, D), jnp.float32),
            pltpu.SemaphoreType.DMA((_NBUF,)),
            pltpu.SemaphoreType.DMA((_NBUF,)),
            pltpu.SemaphoreType.DMA((2, B)),
        ],
    )(qf, k, v)
    return out.reshape(B, N, D)
